# baseline (speedup 1.0000x reference)
import functools

import jax
import jax.numpy as jnp
from jax import lax
from jax.experimental import pallas as pl
from jax.experimental.pallas import tpu as pltpu
from jax.experimental.pallas import tpu_sc as plsc

_N = 10000
_M = 2500
_MP = 2560
_NL = 1264
_NPAD = 8 * _NL
_ML = 320
_K = 64
_R1SQ = 0.2 * 0.2
_R2SQ = 0.4 * 0.4
_NW = 32
_RPW = _MP // _NW
_NCH = _NPAD // 16
_CAP = 1008
_EPS = 1e-5
_RB = 128
_EB = _RB * _K
_GRID = _MP // _RB


def _fps_body(posp_ref, idx_ref, posq_ref, dists_ref):
    px = posp_ref[0]
    py = posp_ref[1]
    pz = posp_ref[2]
    iota2d = (jax.lax.broadcasted_iota(jnp.int32, (8, _NL), 0) * _NL
              + jax.lax.broadcasted_iota(jnp.int32, (8, _NL), 1))
    valid = iota2d < _N
    dists_ref[...] = jnp.where(valid, jnp.inf, -jnp.inf).astype(jnp.float32)
    siota = (jax.lax.broadcasted_iota(jnp.int32, (8, _ML), 0) * _ML
             + jax.lax.broadcasted_iota(jnp.int32, (8, _ML), 1))

    c0x, c0y, c0z = posp_ref[0, 0, 0], posp_ref[1, 0, 0], posp_ref[2, 0, 0]
    m0 = siota == 0
    idx_ref[...] = jnp.where(m0, 0, 0).astype(jnp.int32)
    posq_ref[0] = jnp.where(m0, c0x, 0.0)
    posq_ref[1] = jnp.where(m0, c0y, 0.0)
    posq_ref[2] = jnp.where(m0, c0z, 0.0)

    def body(i, c):
        cx, cy, cz = c
        dx = px - cx
        dy = py - cy
        dz = pz - cz
        d = dx * dx + dy * dy + dz * dz
        dn = jnp.minimum(dists_ref[...], d)
        dists_ref[...] = dn
        mx = jnp.max(dn)
        nxt = jnp.min(jnp.where(dn == mx, iota2d, jnp.int32(2 ** 30)))
        hit = iota2d == nxt
        ninf = jnp.float32(-jnp.inf)
        nx = jnp.max(jnp.where(hit, px, ninf))
        ny = jnp.max(jnp.where(hit, py, ninf))
        nz = jnp.max(jnp.where(hit, pz, ninf))
        ms = siota == i
        idx_ref[...] = jnp.where(ms, nxt, idx_ref[...])
        posq_ref[0] = jnp.where(ms, nx, posq_ref[0])
        posq_ref[1] = jnp.where(ms, ny, posq_ref[1])
        posq_ref[2] = jnp.where(ms, nz, posq_ref[2])
        return (nx, ny, nz)

    jax.lax.fori_loop(1, _M, body, (c0x, c0y, c0z))


def _fps_pallas(posp_flat):
    idx8, posq8 = pl.pallas_call(
        _fps_body,
        out_shape=[
            jax.ShapeDtypeStruct((8, _ML), jnp.int32),
            jax.ShapeDtypeStruct((3, 8, _ML), jnp.float32),
        ],
        scratch_shapes=[pltpu.VMEM((8, _NL), jnp.float32)],
    )(posp_flat.reshape(3, 8, _NL))
    idx = idx8.reshape(-1)[:_M]
    posqT = posq8.reshape(3, _MP)
    return idx, posqT


def _proj_body(x_ref, p8_ref, q8_ref, wx1_ref, wp1_ref, wx2_ref, wp2_ref,
               a1_ref, v1_ref, v2_ref):
    f32 = jnp.float32
    x = x_ref[...]
    p8 = p8_ref[...]
    q8 = q8_ref[...]
    a1 = (jnp.dot(x, wx1_ref[...], preferred_element_type=f32)
          + jnp.dot(p8, wp1_ref[...], preferred_element_type=f32))
    a2 = (jnp.dot(x, wx2_ref[...], preferred_element_type=f32)
          + jnp.dot(p8, wp2_ref[...], preferred_element_type=f32))
    a1_ref[...] = jnp.concatenate(
        [a1, a2, jnp.zeros((_N, 32), f32)], axis=1)
    v1_ref[...] = jnp.dot(q8, wp1_ref[...], preferred_element_type=f32)
    v2_ref[...] = jnp.dot(q8, wp2_ref[...], preferred_element_type=f32)


def _proj_pallas(x, p8, q8, wx1, wp1, wx2, wp2):
    return pl.pallas_call(
        _proj_body,
        out_shape=[
            jax.ShapeDtypeStruct((_N, 128), jnp.float32),
            jax.ShapeDtypeStruct((_MP, 32), jnp.float32),
            jax.ShapeDtypeStruct((_MP, 64), jnp.float32),
        ],
    )(x, p8, q8, wx1, wp1, wx2, wp2)


_DUMP = _MP * _K
_NB = _NPAD // 128
_BISECT = 42


def _select_body(q8_ref, posp8_ref, tgt_ref, pay_ref):
    f32 = jnp.float32
    i = pl.program_id(0)
    q8 = q8_ref[...]
    posp8 = posp8_ref[...]
    dx = q8[:, 0:1] - posp8[0:1, :]
    dy = q8[:, 1:2] - posp8[1:2, :]
    dz = q8[:, 2:3] - posp8[2:3, :]
    d2 = dx * dx + dy * dy + dz * dz
    rsub = jax.lax.broadcasted_iota(jnp.int32, (_RB, 1), 0) + i * _RB
    d2 = jnp.where(rsub < _M, d2, jnp.inf)

    ones8 = jnp.ones((_NPAD, 8), f32)

    def bis(_, lohi):
        lo, hi = lohi
        mid = (lo + hi) * 0.5
        cnt = jnp.dot((d2 <= mid).astype(f32), ones8,
                      preferred_element_type=f32)[:, 0:1]
        ge = cnt >= jnp.float32(_K)
        return (jnp.where(ge, lo, mid), jnp.where(ge, mid, hi))

    lo, hi = jax.lax.fori_loop(
        0, _BISECT, bis,
        (jnp.zeros((_RB, 1), f32), jnp.full((_RB, 1), jnp.float32(_R2SQ), f32)))

    sel = d2 <= hi
    self_f = sel.astype(f32)
    li = jax.lax.broadcasted_iota(jnp.int32, (128, 128), 0)
    lj = jax.lax.broadcasted_iota(jnp.int32, (128, 128), 1)
    lt = (li < lj).astype(f32)
    run = jnp.zeros((_RB, 1), f32)
    parts = []
    for b in range(_NB):
        sblk = self_f[:, b * 128:(b + 1) * 128]
        pwb = jnp.dot(sblk, lt, preferred_element_type=f32)
        parts.append(pwb + run)
        run = run + pwb[:, 127:128] + sblk[:, 127:128]
    rank = jnp.concatenate(parts, axis=1)
    ranki = jnp.minimum(rank.astype(jnp.int32), _K - 1)

    tgt = jnp.where(sel, rsub * _K + ranki,
                    _DUMP + jnp.bitwise_and(jax.lax.broadcasted_iota(
                        jnp.int32, (_RB, _NPAD), 1), 127))
    jiota = jax.lax.broadcasted_iota(jnp.int32, (_RB, _NPAD), 1)
    b1 = jnp.logical_and(d2 <= jnp.float32(_R1SQ), sel)
    pay = (jiota + jnp.where(b1, 16384, 0)
           + jnp.where(sel, 32768, 0))
    tgt_ref[...] = tgt
    pay_ref[...] = pay


def _select_pallas(q8, posp8):
    return pl.pallas_call(
        _select_body,
        grid=(_GRID,),
        in_specs=[
            pl.BlockSpec((_RB, 8), lambda i: (i, 0)),
            pl.BlockSpec((8, _NPAD), lambda i: (0, 0)),
        ],
        out_specs=[
            pl.BlockSpec((_RB, _NPAD), lambda i: (i, 0)),
            pl.BlockSpec((_RB, _NPAD), lambda i: (i, 0)),
        ],
        out_shape=[
            jax.ShapeDtypeStruct((_MP, _NPAD), jnp.int32),
            jax.ShapeDtypeStruct((_MP, _NPAD), jnp.int32),
        ],
    )(q8, posp8)


def _sc_compact_gather(tgt, pay, ac):
    mesh = plsc.VectorSubcoreMesh(core_axis_name="c", subcore_axis_name="s")

    @functools.partial(
        pl.kernel, mesh=mesh,
        out_type=[
            jax.ShapeDtypeStruct((_MP * _K + 128,), jnp.int32),
            jax.ShapeDtypeStruct((_MP * _K, 128), jnp.float32),
        ],
        scratch_types=[
            pltpu.VMEM((_NB + 1, 128), jnp.int32),
            pltpu.VMEM((_NB + 1, 128), jnp.int32),
            pltpu.VMEM((1024,), jnp.int32),
            pltpu.VMEM((_K,), jnp.int32),
            pltpu.VMEM((_K,), jnp.int32),
            pltpu.VMEM((_K, 128), jnp.float32),
            pltpu.SemaphoreType.DMA,
        ],
    )
    def sck(tgt_h, pay_h, ac_h, enc_h, gc_h,
            tbuf, pbuf, zbuf, ebuf, ibuf, gcrow, sem1):
        wid = lax.axis_index("s") * 2 + lax.axis_index("c")
        base = wid * _RPW
        z16 = jnp.zeros((16,), jnp.int32)

        def zinit(c, carry):
            zbuf[pl.ds(c * 16, 16)] = z16
            return carry

        lax.fori_loop(0, 64, zinit, 0)
        def pre(k, carry):
            pltpu.sync_copy(zbuf, enc_h.at[pl.ds((base + k * 16) * _K, 1024)])
            return carry

        lax.fori_loop(0, _RPW // 16, pre, 0)

        def row_body(rl, carry):
            r = base + rl
            nb = _NB + 1
            pltpu.sync_copy(tgt_h.at[pl.ds(r * nb, nb)], tbuf)
            pltpu.sync_copy(pay_h.at[pl.ds(r * nb, nb)], pbuf)

            def sc_chunk(c, carry2):
                pltpu.async_copy(pbuf.at[c], enc_h.at[tbuf.at[c]], sem1).wait()
                return carry2

            lax.fori_loop(0, nb, sc_chunk, 0)
            pltpu.sync_copy(enc_h.at[pl.ds(r * _K, _K)], ebuf)

            def dec(c, carry):
                e = ebuf[pl.ds(c * 16, 16)]
                ibuf[pl.ds(c * 16, 16)] = jnp.bitwise_and(e, 16383)
                return carry

            lax.fori_loop(0, _K // 16, dec, 0)
            pltpu.async_copy(ac_h.at[ibuf], gcrow, sem1).wait()
            pltpu.sync_copy(gcrow, gc_h.at[pl.ds(r * _K, _K)])
            return carry

        lax.fori_loop(0, _RPW, row_body, jnp.int32(0))

    tgtp = jnp.pad(tgt, ((0, 0), (0, 128)), constant_values=_DUMP)
    payp = jnp.pad(pay, ((0, 0), (0, 128)))
    return sck(tgtp.reshape(_MP * (_NB + 1), 128),
               payp.reshape(_MP * (_NB + 1), 128), ac)


def _stats_body(gc_ref, d2e_ref, v1_ref, v2_ref,
                s1_ref, q1_ref, n1_ref, s2_ref, q2_ref, n2_ref):
    i = pl.program_id(0)

    @pl.when(i == 0)
    def _():
        s1_ref[...] = jnp.zeros_like(s1_ref)
        q1_ref[...] = jnp.zeros_like(q1_ref)
        n1_ref[...] = jnp.zeros_like(n1_ref)
        s2_ref[...] = jnp.zeros_like(s2_ref)
        q2_ref[...] = jnp.zeros_like(q2_ref)
        n2_ref[...] = jnp.zeros_like(n2_ref)

    enc = d2e_ref[...]
    gc = gc_ref[...]
    v1e = jnp.broadcast_to(v1_ref[...][:, None, :],
                           (_RB, _K, 32)).reshape(_EB, 32)
    h1 = gc[:, 0:32] - v1e
    m1 = (jnp.bitwise_and(enc, 16384) > 0).astype(jnp.float32)
    hm1 = h1 * m1
    s1_ref[...] += jnp.broadcast_to(jnp.sum(hm1, axis=0)[None, :], (8, 32))
    q1_ref[...] += jnp.broadcast_to(jnp.sum(hm1 * h1, axis=0)[None, :], (8, 32))
    n1_ref[...] += jnp.sum(m1)

    v2e = jnp.broadcast_to(v2_ref[...][:, None, :],
                           (_RB, _K, 64)).reshape(_EB, 64)
    h2 = gc[:, 32:96] - v2e
    m2 = (jnp.bitwise_and(enc, 32768) > 0).astype(jnp.float32)
    hm2 = h2 * m2
    s2_ref[...] += jnp.broadcast_to(jnp.sum(hm2, axis=0)[None, :], (8, 64))
    q2_ref[...] += jnp.broadcast_to(jnp.sum(hm2 * h2, axis=0)[None, :], (8, 64))
    n2_ref[...] += jnp.sum(m2)


def _stats_pallas(gc, d2e, v1, v2):
    return pl.pallas_call(
        _stats_body,
        grid=(_GRID,),
        in_specs=[
            pl.BlockSpec((_EB, 128), lambda i: (i, 0)),
            pl.BlockSpec((_EB, 1), lambda i: (i, 0)),
            pl.BlockSpec((_RB, 32), lambda i: (i, 0)),
            pl.BlockSpec((_RB, 64), lambda i: (i, 0)),
        ],
        out_specs=[
            pl.BlockSpec((8, 32), lambda i: (0, 0)),
            pl.BlockSpec((8, 32), lambda i: (0, 0)),
            pl.BlockSpec((8, 128), lambda i: (0, 0)),
            pl.BlockSpec((8, 64), lambda i: (0, 0)),
            pl.BlockSpec((8, 64), lambda i: (0, 0)),
            pl.BlockSpec((8, 128), lambda i: (0, 0)),
        ],
        out_shape=[
            jax.ShapeDtypeStruct((8, 32), jnp.float32),
            jax.ShapeDtypeStruct((8, 32), jnp.float32),
            jax.ShapeDtypeStruct((8, 128), jnp.float32),
            jax.ShapeDtypeStruct((8, 64), jnp.float32),
            jax.ShapeDtypeStruct((8, 64), jnp.float32),
            jax.ShapeDtypeStruct((8, 128), jnp.float32),
        ],
    )(gc, d2e, v1, v2)


def _apply_body(gc_ref, d2e_ref, v1_ref, v2_ref, w11_ref, w21_ref,
                b1_ref, b2_ref, s1_ref, q1_ref, n1_ref, s2_ref, q2_ref, n2_ref,
                o1_ref, o2_ref):
    f32 = jnp.float32
    enc = d2e_ref[...]
    ninf = jnp.float32(-jnp.inf)

    n1 = n1_ref[0, 0]
    mu1 = s1_ref[...][0:1, :] / n1
    var1 = q1_ref[...][0:1, :] / n1 - mu1 * mu1
    isd1 = lax.rsqrt(var1 + _EPS)
    v1e = jnp.broadcast_to(v1_ref[...][:, None, :],
                           (_RB, _K, 32)).reshape(_EB, 32)
    gc = gc_ref[...]
    hn1 = jnp.maximum((gc[:, 0:32] - v1e - mu1) * isd1, 0.0)
    u1 = jnp.dot(hn1, w11_ref[...], preferred_element_type=f32)
    u1m = jnp.where(jnp.bitwise_and(enc, 16384) > 0, u1, ninf)
    o1 = jnp.max(u1m.reshape(_RB, _K, 64), axis=1)
    o1_ref[...] = o1 + b1_ref[...][0:1, :]

    n2 = n2_ref[0, 0]
    mu2 = s2_ref[...][0:1, :] / n2
    var2 = q2_ref[...][0:1, :] / n2 - mu2 * mu2
    isd2 = lax.rsqrt(var2 + _EPS)
    v2e = jnp.broadcast_to(v2_ref[...][:, None, :],
                           (_RB, _K, 64)).reshape(_EB, 64)
    hn2 = jnp.maximum((gc[:, 32:96] - v2e - mu2) * isd2, 0.0)
    u2 = jnp.dot(hn2, w21_ref[...], preferred_element_type=f32)
    u2m = jnp.where(jnp.bitwise_and(enc, 32768) > 0, u2, ninf)
    o2 = jnp.max(u2m.reshape(_RB, _K, 128), axis=1)
    o2_ref[...] = o2 + b2_ref[...][0:1, :]


def _apply_pallas(gc, d2e, v1, v2, w11, w21, b1, b2,
                  s1, q1, n1, s2, q2, n2):
    cst = lambda i: (0, 0)
    return pl.pallas_call(
        _apply_body,
        grid=(_GRID,),
        in_specs=[
            pl.BlockSpec((_EB, 128), lambda i: (i, 0)),
            pl.BlockSpec((_EB, 1), lambda i: (i, 0)),
            pl.BlockSpec((_RB, 32), lambda i: (i, 0)),
            pl.BlockSpec((_RB, 64), lambda i: (i, 0)),
            pl.BlockSpec((32, 64), cst),
            pl.BlockSpec((64, 128), cst),
            pl.BlockSpec((8, 64), cst),
            pl.BlockSpec((8, 128), cst),
            pl.BlockSpec((8, 32), cst),
            pl.BlockSpec((8, 32), cst),
            pl.BlockSpec((8, 128), cst),
            pl.BlockSpec((8, 64), cst),
            pl.BlockSpec((8, 64), cst),
            pl.BlockSpec((8, 128), cst),
        ],
        out_specs=[
            pl.BlockSpec((_RB, 64), lambda i: (i, 0)),
            pl.BlockSpec((_RB, 128), lambda i: (i, 0)),
        ],
        out_shape=[
            jax.ShapeDtypeStruct((_MP, 64), jnp.float32),
            jax.ShapeDtypeStruct((_MP, 128), jnp.float32),
        ],
    )(gc, d2e, v1, v2, w11, w21, b1, b2, s1, q1, n1, s2, q2, n2)


def kernel(x, pos, batch, w10, b10, w11, b11, w20, b20, w21, b21):
    f32 = jnp.float32
    posT = jnp.transpose(pos)
    posp_flat = jnp.pad(posT, ((0, 0), (0, _NPAD - _N)),
                        constant_values=1e9)
    idx, posqT = _fps_pallas(posp_flat)
    pos_q = posqT[:, :_M].T

    p8 = jnp.concatenate([pos, jnp.ones((_N, 1), f32),
                          jnp.zeros((_N, 4), f32)], axis=1)
    q8 = jnp.concatenate([posqT.T, jnp.zeros((_MP, 5), f32)], axis=1)
    wp1 = jnp.concatenate([w10[64:67], b10[None, :],
                           jnp.zeros((4, 32), f32)], axis=0)
    wp2 = jnp.concatenate([w20[64:67], b20[None, :],
                           jnp.zeros((4, 64), f32)], axis=0)
    ac, v1, v2 = _proj_pallas(x, p8, q8, w10[:64], wp1, w20[:64], wp2)

    posp8 = jnp.concatenate([posp_flat, jnp.zeros((5, _NPAD), f32)], axis=0)
    tgt, pay = _select_pallas(q8, posp8)
    enc_all = jnp.zeros((_MP * _K + 128,), jnp.int32)
    enc_all = enc_all.at[tgt.reshape(-1)].set(pay.reshape(-1))
    gc = ac[jnp.bitwise_and(enc_all[:_MP * _K], 16383)]
    d2e = enc_all[:_MP * _K].reshape(_MP * _K, 1)

    s1, q1, n1, s2, q2, n2 = _stats_pallas(gc, d2e, v1, v2)
    b1 = jnp.broadcast_to(b11, (8, 64))
    b2 = jnp.broadcast_to(b21, (8, 128))
    o1, o2 = _apply_pallas(gc, d2e, v1, v2, w11, w21, b1, b2,
                           s1, q1, n1, s2, q2, n2)
    new_x = jnp.concatenate([o1[:_M], o2[:_M]], axis=1)
    return new_x, pos_q, jnp.take(batch, idx)
